# Initial kernel scaffold; baseline (speedup 1.0000x reference)
import jax
import jax.numpy as jnp
from jax.experimental import pallas as pl


def kernel(indices, embedding):
    raise NotImplementedError("write your pallas kernel here")


# speedup vs baseline: 1.1116x; 1.1116x over previous
import jax
import jax.numpy as jnp
from jax import lax
from jax.experimental import pallas as pl
from jax.experimental.pallas import tpu as pltpu
from jax.experimental.pallas import tpu_sc as plsc

E = 4
V = 1_000_000
D = 16
NW = 32
N = 16384 * 26
PER_W = N // NW
G = 1664
NCH = PER_W // G


def _sc_body(idx_hbm, emb_hbm, out_hbm, idx_v, buf0, buf1, sem0, sem1):
    wid = lax.axis_index("s") * 2 + lax.axis_index("c")
    base = wid * PER_W
    pltpu.sync_copy(idx_hbm.at[pl.ds(base, PER_W)], idx_v)

    bufs = (buf0, buf1)
    sems = (sem0, sem1)
    chunks = [(e, c) for e in range(E) for c in range(NCH)]

    def issue(k):
        e, c = chunks[k]
        pltpu.async_copy(
            emb_hbm.at[e].at[idx_v.at[pl.ds(c * G, G)]],
            bufs[k % 2],
            sems[k % 2],
        )

    issue(0)
    for k in range(len(chunks)):
        if k + 1 < len(chunks):
            issue(k + 1)
        e, c = chunks[k]
        pltpu.make_async_copy(
            emb_hbm.at[e].at[idx_v.at[pl.ds(c * G, G)]],
            bufs[k % 2],
            sems[k % 2],
        ).wait()
        pltpu.sync_copy(bufs[k % 2], out_hbm.at[e, pl.ds(base + c * G, G)])


def _lookup(idx_flat, embedding):
    mesh = plsc.VectorSubcoreMesh(core_axis_name="c", subcore_axis_name="s")
    return pl.kernel(
        _sc_body,
        out_type=jax.ShapeDtypeStruct((E, N, D), jnp.float32),
        mesh=mesh,
        scratch_types=[
            pltpu.VMEM((PER_W,), jnp.int32),
            pltpu.VMEM((G, D), jnp.float32),
            pltpu.VMEM((G, D), jnp.float32),
            pltpu.SemaphoreType.DMA,
            pltpu.SemaphoreType.DMA,
        ],
        compiler_params=pltpu.CompilerParams(use_tc_tiling_on_sc=False),
    )(idx_flat, embedding)


def kernel(indices, embedding):
    b, f = indices.shape
    out = _lookup(indices.reshape(-1), embedding)
    return out.reshape(E, b, f, D)
